# baseline (speedup 1.0000x reference)
import functools

import jax
import jax.numpy as jnp
from jax import lax
from jax.experimental import pallas as pl
from jax.experimental.pallas import tpu as pltpu
from jax.experimental.pallas import tpu_sc as plsc

CHUNK = 100


def _make_sc_kernel(B, S, D, interpret=False):
    n_chunks = (B * S) // CHUNK
    nc, ns = 2, 16
    nw = nc * ns
    chunks_per_w = n_chunks // nw
    per_seq = S // CHUNK
    mesh = plsc.VectorSubcoreMesh(core_axis_name="c", subcore_axis_name="s",
                                  num_cores=nc, num_subcores=ns)

    @functools.partial(
        pl.kernel,
        out_type=jax.ShapeDtypeStruct((n_chunks, CHUNK, D), jnp.float32),
        mesh=mesh,
        scratch_types=[
            pltpu.VMEM((chunks_per_w, CHUNK), jnp.int32),
            pltpu.VMEM_SHARED((per_seq, CHUNK, D), jnp.float32),
            pltpu.VMEM((CHUNK, D), jnp.float32),
            pltpu.SemaphoreType.DMA,
        ],
        compiler_params=pltpu.CompilerParams(use_tc_tiling_on_sc=False),
        interpret=interpret,
    )
    def k(idx_hbm, table_hbm, pos_hbm, out_hbm, idx_v, pos_sh, dest_v, sem):
        sid = lax.axis_index("s")
        wid = sid * nc + lax.axis_index("c")
        base = wid * chunks_per_w
        pltpu.sync_copy(idx_hbm.at[pl.ds(base, chunks_per_w)], idx_v)

        @pl.when(sid == 0)
        def _fill_pos():
            pltpu.sync_copy(pos_hbm, pos_sh)

        plsc.subcore_barrier()

        def body(c, carry):
            pltpu.sync_copy(pos_sh.at[lax.rem(c, per_seq)], dest_v)
            pltpu.async_copy(table_hbm.at[idx_v.at[c]], dest_v, sem,
                             add=True).wait()
            pltpu.sync_copy(dest_v, out_hbm.at[base + c])
            return carry

        lax.fori_loop(0, chunks_per_w, body, 0)

    return k


def kernel(x, emb_table, pos_emb):
    B, S = x.shape
    D = emb_table.shape[1]
    idx = x.reshape(-1, CHUNK).astype(jnp.int32)
    pos = pos_emb[:S].reshape(S // CHUNK, CHUNK, D)
    out = _make_sc_kernel(B, S, D)(idx, emb_table, pos)
    return out.reshape(B, S, D)


# speedup vs baseline: 1.0750x; 1.0750x over previous
import functools

import jax
import jax.numpy as jnp
from jax import lax
from jax.experimental import pallas as pl
from jax.experimental.pallas import tpu as pltpu
from jax.experimental.pallas import tpu_sc as plsc

CHUNK = 100
GROUP = 4
NBUF = 3


def _make_sc_kernel(B, S, D):
    n_chunks = (B * S) // CHUNK
    nc, ns = 2, 16
    nw = nc * ns
    chunks_per_w = n_chunks // nw
    n_groups = chunks_per_w // GROUP
    mesh = plsc.VectorSubcoreMesh(core_axis_name="c", subcore_axis_name="s",
                                  num_cores=nc, num_subcores=ns)

    @functools.partial(
        pl.kernel,
        out_type=jax.ShapeDtypeStruct((n_chunks, CHUNK, D), jnp.float32),
        mesh=mesh,
        scratch_types=[
            pltpu.VMEM((chunks_per_w, CHUNK), jnp.int32),
            pltpu.VMEM_SHARED((GROUP, CHUNK, D), jnp.float32),
            pltpu.VMEM((NBUF, GROUP, CHUNK, D), jnp.float32),
            [pltpu.SemaphoreType.DMA] * NBUF,
            [pltpu.SemaphoreType.DMA] * NBUF,
            [pltpu.SemaphoreType.DMA] * NBUF,
        ],
        compiler_params=pltpu.CompilerParams(use_tc_tiling_on_sc=False),
    )
    def k(idx_hbm, table_hbm, pos_hbm, out_hbm, idx_v, pos_sh, dest,
          sem_pre, sem_gat, sem_wb):
        sid = lax.axis_index("s")
        wid = sid * nc + lax.axis_index("c")
        base = wid * chunks_per_w
        pltpu.sync_copy(idx_hbm.at[pl.ds(base, chunks_per_w)], idx_v)

        @pl.when(sid == 0)
        def _fill_pos():
            pltpu.sync_copy(pos_hbm, pos_sh)

        plsc.subcore_barrier()

        def gathers(g, b):
            return [
                pltpu.make_async_copy(
                    table_hbm.at[idx_v.at[g * GROUP + j]],
                    dest.at[b, j], sem_gat[b])
                for j in range(GROUP)
            ]

        def writeback(g, b):
            return pltpu.make_async_copy(
                dest.at[b], out_hbm.at[pl.ds(base + g * GROUP, GROUP)],
                sem_wb[b])

        def step(t, b):
            @pl.when(jnp.logical_and(t >= NBUF, t < n_groups))
            def _wait_wb():
                writeback(t - NBUF, b).wait()

            @pl.when(t < n_groups)
            def _pre():
                pltpu.make_async_copy(pos_sh, dest.at[b], sem_pre[b]).start()

            bg = (b - 1) % NBUF

            @pl.when(jnp.logical_and(t - 1 >= 0, t - 1 < n_groups))
            def _gat():
                pltpu.make_async_copy(pos_sh, dest.at[bg], sem_pre[bg]).wait()
                for d in gathers(t - 1, bg):
                    d.start(add=True)

            bw = (b - 2) % NBUF

            @pl.when(jnp.logical_and(t - 2 >= 0, t - 2 < n_groups))
            def _wb():
                for d in gathers(t - 2, bw):
                    d.wait()
                writeback(t - 2, bw).start()

        n_steps = n_groups + 2
        n_outer = -(-n_steps // NBUF)

        def outer(i, carry):
            for b in range(NBUF):
                step(i * NBUF + b, b)
            return carry

        lax.fori_loop(0, n_outer, outer, 0)

        for g in range(n_groups - min(NBUF, n_groups), n_groups):
            writeback(g, g % NBUF).wait()

    return k


def kernel(x, emb_table, pos_emb):
    B, S = x.shape
    D = emb_table.shape[1]
    idx = x.reshape(-1, CHUNK).astype(jnp.int32)
    pos = jnp.tile(pos_emb[:S], ((GROUP * CHUNK) // S, 1)).reshape(
        GROUP, CHUNK, D)
    out = _make_sc_kernel(B, S, D)(idx, emb_table, pos)
    return out.reshape(B, S, D)
